# baseline (speedup 1.0000x reference)
import functools
import math

import jax
import jax.numpy as jnp
from jax import lax
from jax.experimental import pallas as pl
from jax.experimental.pallas import tpu as pltpu
from jax.experimental.pallas import tpu_sc as plsc

N_PREDICT = 3
EMB_DIM = 4096
INNER_DIM = 1024
VOCAB = 32000
SEQ = 2048
STATE_WEIGHT = 0.5 ** (0.5 / N_PREDICT)
EMB_WEIGHT = math.sqrt((1.0 - STATE_WEIGHT ** 2) * (INNER_DIM / 2.0))
ALPHA = EMB_WEIGHT / STATE_WEIGHT
EPS = 1e-06

_RB = 256
_VB = 1280


def _rms(x):
    return x * jax.lax.rsqrt(jnp.mean(x * x, axis=-1, keepdims=True) + EPS)


def _gelu(x):
    return x * 0.5 * (1.0 + jax.lax.erf(x * (0.5 ** 0.5)))


def _proj0_kernel(state_ref, p0_ref, m0_ref, p0_bf):
    @pl.when(pl.program_id(0) == 0)
    def _cast_weights():
        p0_bf[...] = p0_ref[...].astype(jnp.bfloat16)

    x = state_ref[0]
    s = _rms(x) * (0.5 ** 0.5)
    m0_ref[...] = jax.lax.dot_general(s.astype(jnp.bfloat16), p0_bf[...],
                                      (((1,), (1,)), ((), ())),
                                      preferred_element_type=jnp.float32)


def _h0_kernel(m0_ref, z0_ref, lnw_ref, lnb_ref, h0_ref):
    t = m0_ref[...] + ALPHA * z0_ref[...]
    t = _rms(t) * lnw_ref[0][None, :] + lnb_ref[0][None, :]
    h0_ref[...] = _gelu(t).astype(jnp.bfloat16)


def _logits_kernel(h0_hbm, zr_hbm, pr_hbm, lnw_ref, lnb_ref, w_ref, out_ref,
                   hb_ref, zsc_ref, prsc_ref, sem_h, sem_z, sem_p):
    h = pl.program_id(0)
    v = pl.program_id(1)
    nv = pl.num_programs(1)

    @pl.when((h == 0) & (v == 0))
    def _init():
        cp = pltpu.make_async_copy(h0_hbm, hb_ref, sem_h)
        cp.start()
        cp.wait()

    @pl.when((h > 0) & (v == 0))
    def _chain():
        pltpu.make_async_copy(zr_hbm.at[h - 1], zsc_ref, sem_z).wait()
        pltpu.make_async_copy(pr_hbm.at[h - 1], prsc_ref, sem_p).wait()
        pw = prsc_ref[...].astype(jnp.bfloat16)
        lw = lnw_ref[pl.ds(h, 1)][0][None, :]
        lb = lnb_ref[pl.ds(h, 1)][0][None, :]
        for r0 in range(0, SEQ, 512):
            rows = pl.ds(r0, 512)
            m = jax.lax.dot_general(hb_ref[rows, :], pw,
                                    (((1,), (1,)), ((), ())),
                                    preferred_element_type=jnp.float32)
            t = m + ALPHA * zsc_ref[rows, :]
            t = _rms(t) * lw + lb
            hb_ref[rows, :] = _gelu(t).astype(jnp.bfloat16)

    w = w_ref[0].astype(jnp.bfloat16)
    out_ref[0, 0] = jax.lax.dot_general(
        hb_ref[...], w, (((1,), (1,)), ((), ())),
        preferred_element_type=jnp.float32)

    @pl.when((v == nv - 1) & (h < N_PREDICT - 1))
    def _prefetch():
        pltpu.make_async_copy(zr_hbm.at[h], zsc_ref, sem_z).start()
        pltpu.make_async_copy(pr_hbm.at[h], prsc_ref, sem_p).start()


def _sc_gather(emb_flat, idx_all, n):
    info = plsc.get_sparse_core_info()
    nc, ns = info.num_cores, info.num_subcores
    nw = nc * ns
    rows_w = n // nw

    mesh = plsc.VectorSubcoreMesh(core_axis_name="c", subcore_axis_name="s")

    @functools.partial(
        pl.kernel,
        out_type=(
            jax.ShapeDtypeStruct((n, INNER_DIM), jnp.float32),
            jax.ShapeDtypeStruct((N_PREDICT - 1, n, INNER_DIM), jnp.float32),
        ),
        mesh=mesh,
        scratch_types=[
            pltpu.VMEM((rows_w,), jnp.int32),
            pltpu.VMEM((rows_w, INNER_DIM), jnp.float32),
            pltpu.SemaphoreType.DMA,
        ],
    )
    def gather_k(table_hbm, idx_hbm, z0_hbm, zr_hbm, idx_v, rows_v, sem):
        wid = lax.axis_index("s") * nc + lax.axis_index("c")
        for i in range(N_PREDICT):
            pltpu.sync_copy(idx_hbm.at[pl.ds(i * n + wid * rows_w, rows_w)],
                            idx_v)
            pltpu.async_copy(table_hbm.at[idx_v], rows_v, sem).wait()
            if i == 0:
                pltpu.sync_copy(rows_v, z0_hbm.at[pl.ds(wid * rows_w, rows_w)])
            else:
                pltpu.sync_copy(
                    rows_v, zr_hbm.at[i - 1].at[pl.ds(wid * rows_w, rows_w)])

    return gather_k(emb_flat, idx_all)


def kernel(state, inds, emb_w, proj0_w, proj_rest_w, head_w, ln_w, ln_b):
    n = state.shape[1]
    idx_all = jnp.concatenate(
        [inds[0, i:i + n] + i * VOCAB for i in range(N_PREDICT)])
    z0, z_rest = _sc_gather(emb_w.reshape(N_PREDICT * VOCAB, INNER_DIM),
                            idx_all, n)

    nr = n // _RB
    m0 = pl.pallas_call(
        _proj0_kernel,
        grid=(nr,),
        in_specs=[
            pl.BlockSpec((1, _RB, EMB_DIM), lambda r: (0, r, 0)),
            pl.BlockSpec((INNER_DIM, EMB_DIM), lambda r: (0, 0)),
        ],
        out_specs=pl.BlockSpec((_RB, INNER_DIM), lambda r: (r, 0)),
        out_shape=jax.ShapeDtypeStruct((n, INNER_DIM), jnp.float32),
        scratch_shapes=[pltpu.VMEM((INNER_DIM, EMB_DIM), jnp.bfloat16)],
        compiler_params=pltpu.CompilerParams(vmem_limit_bytes=100 * 2 ** 20),
    )(state, proj0_w)

    h0 = pl.pallas_call(
        _h0_kernel,
        grid=(nr,),
        in_specs=[
            pl.BlockSpec((_RB, INNER_DIM), lambda r: (r, 0)),
            pl.BlockSpec((_RB, INNER_DIM), lambda r: (r, 0)),
            pl.BlockSpec((N_PREDICT, INNER_DIM), lambda r: (0, 0)),
            pl.BlockSpec((N_PREDICT, INNER_DIM), lambda r: (0, 0)),
        ],
        out_specs=pl.BlockSpec((_RB, INNER_DIM), lambda r: (r, 0)),
        out_shape=jax.ShapeDtypeStruct((n, INNER_DIM), jnp.bfloat16),
        compiler_params=pltpu.CompilerParams(vmem_limit_bytes=100 * 2 ** 20),
    )(m0, z0, ln_w, ln_b)

    nv = VOCAB // _VB
    logits = pl.pallas_call(
        _logits_kernel,
        grid=(N_PREDICT, nv),
        in_specs=[
            pl.BlockSpec(memory_space=pl.ANY),
            pl.BlockSpec(memory_space=pl.ANY),
            pl.BlockSpec(memory_space=pl.ANY),
            pl.BlockSpec((N_PREDICT, INNER_DIM), lambda h, v: (0, 0)),
            pl.BlockSpec((N_PREDICT, INNER_DIM), lambda h, v: (0, 0)),
            pl.BlockSpec((1, _VB, INNER_DIM), lambda h, v: (h, v, 0)),
        ],
        out_specs=pl.BlockSpec((1, 1, n, _VB), lambda h, v: (h, 0, 0, v)),
        out_shape=jax.ShapeDtypeStruct((N_PREDICT, 1, n, VOCAB), jnp.float32),
        scratch_shapes=[
            pltpu.VMEM((n, INNER_DIM), jnp.bfloat16),
            pltpu.VMEM((n, INNER_DIM), jnp.float32),
            pltpu.VMEM((INNER_DIM, INNER_DIM), jnp.float32),
            pltpu.SemaphoreType.DMA,
            pltpu.SemaphoreType.DMA,
            pltpu.SemaphoreType.DMA,
        ],
        compiler_params=pltpu.CompilerParams(vmem_limit_bytes=100 * 2 ** 20),
    )(h0, z_rest, proj_rest_w, ln_w, ln_b, head_w)
    return logits


# speedup vs baseline: 1.0092x; 1.0092x over previous
import functools
import math

import jax
import jax.numpy as jnp
from jax import lax
from jax.experimental import pallas as pl
from jax.experimental.pallas import tpu as pltpu
from jax.experimental.pallas import tpu_sc as plsc

N_PREDICT = 3
EMB_DIM = 4096
INNER_DIM = 1024
VOCAB = 32000
SEQ = 2048
STATE_WEIGHT = 0.5 ** (0.5 / N_PREDICT)
EMB_WEIGHT = math.sqrt((1.0 - STATE_WEIGHT ** 2) * (INNER_DIM / 2.0))
ALPHA = EMB_WEIGHT / STATE_WEIGHT
EPS = 1e-06

_RB = 256
_VB = 1280


def _rms(x):
    return x * jax.lax.rsqrt(jnp.mean(x * x, axis=-1, keepdims=True) + EPS)


def _gelu(x):
    return x * 0.5 * (1.0 + jax.lax.erf(x * (0.5 ** 0.5)))


def _proj0_kernel(state_ref, p0_ref, m0_ref, p0_bf):
    @pl.when(pl.program_id(0) == 0)
    def _cast_weights():
        p0_bf[...] = p0_ref[...].astype(jnp.bfloat16)

    x = state_ref[0]
    s = _rms(x) * (0.5 ** 0.5)
    m0_ref[...] = jax.lax.dot_general(s.astype(jnp.bfloat16), p0_bf[...],
                                      (((1,), (1,)), ((), ())),
                                      preferred_element_type=jnp.float32)


def _h0_kernel(m0_ref, z0_ref, lnw_ref, lnb_ref, h0_ref):
    t = m0_ref[...] + ALPHA * z0_ref[...]
    t = _rms(t) * lnw_ref[0][None, :] + lnb_ref[0][None, :]
    h0_ref[...] = _gelu(t).astype(jnp.bfloat16)


def _logits_kernel(h0_hbm, zr_hbm, pr_hbm, lnw_ref, lnb_ref, w_ref, out_ref,
                   hb_ref, zsc_ref, prsc_ref, sem_h, sem_z, sem_p):
    h = pl.program_id(0)
    v = pl.program_id(1)
    nv = pl.num_programs(1)
    nch = 4
    ch = SEQ // nch
    cur = h % 2
    nxt = (h + 1) % 2

    @pl.when((h == 0) & (v == 0))
    def _init():
        cp = pltpu.make_async_copy(h0_hbm, hb_ref.at[0], sem_h)
        cp.start()
        cp.wait()

    w = w_ref[0].astype(jnp.bfloat16)
    hb = hb_ref[pl.ds(cur, 1)][0]
    out_ref[0, 0] = jax.lax.dot_general(
        hb, w, (((1,), (1,)), ((), ())), preferred_element_type=jnp.float32)

    @pl.when((v == nv - nch - 2) & (h < N_PREDICT - 1))
    def _prefetch():
        pltpu.make_async_copy(zr_hbm.at[h], zsc_ref, sem_z).start()
        pltpu.make_async_copy(pr_hbm.at[h], prsc_ref, sem_p).start()

    @pl.when((v == nv - nch) & (h < N_PREDICT - 1))
    def _prefetch_wait():
        pltpu.make_async_copy(zr_hbm.at[h], zsc_ref, sem_z).wait()
        pltpu.make_async_copy(pr_hbm.at[h], prsc_ref, sem_p).wait()

    @pl.when((v >= nv - nch) & (h < N_PREDICT - 1))
    def _chain_chunk():
        c = v - (nv - nch)
        rows = pl.ds(c * ch, ch)
        pw = prsc_ref[...].astype(jnp.bfloat16)
        lw = lnw_ref[pl.ds(h + 1, 1)][0][None, :]
        lb = lnb_ref[pl.ds(h + 1, 1)][0][None, :]
        src = hb_ref[pl.ds(cur, 1), rows, :][0]
        m = jax.lax.dot_general(src, pw, (((1,), (1,)), ((), ())),
                                preferred_element_type=jnp.float32)
        t = m + ALPHA * zsc_ref[rows, :]
        t = _rms(t) * lw + lb
        hb_ref[pl.ds(nxt, 1), rows, :] = _gelu(t).astype(jnp.bfloat16)[None]


def _sc_gather(emb_flat, idx_all, n):
    info = plsc.get_sparse_core_info()
    nc, ns = info.num_cores, info.num_subcores
    nw = nc * ns
    rows_w = n // nw

    mesh = plsc.VectorSubcoreMesh(core_axis_name="c", subcore_axis_name="s")

    @functools.partial(
        pl.kernel,
        out_type=(
            jax.ShapeDtypeStruct((n, INNER_DIM), jnp.float32),
            jax.ShapeDtypeStruct((N_PREDICT - 1, n, INNER_DIM), jnp.float32),
        ),
        mesh=mesh,
        scratch_types=[
            pltpu.VMEM((rows_w,), jnp.int32),
            pltpu.VMEM((rows_w, INNER_DIM), jnp.float32),
            pltpu.SemaphoreType.DMA,
        ],
    )
    def gather_k(table_hbm, idx_hbm, z0_hbm, zr_hbm, idx_v, rows_v, sem):
        wid = lax.axis_index("s") * nc + lax.axis_index("c")
        for i in range(N_PREDICT):
            pltpu.sync_copy(idx_hbm.at[pl.ds(i * n + wid * rows_w, rows_w)],
                            idx_v)
            pltpu.async_copy(table_hbm.at[idx_v], rows_v, sem).wait()
            if i == 0:
                pltpu.sync_copy(rows_v, z0_hbm.at[pl.ds(wid * rows_w, rows_w)])
            else:
                pltpu.sync_copy(
                    rows_v, zr_hbm.at[i - 1].at[pl.ds(wid * rows_w, rows_w)])

    return gather_k(emb_flat, idx_all)


def kernel(state, inds, emb_w, proj0_w, proj_rest_w, head_w, ln_w, ln_b):
    n = state.shape[1]
    idx_all = jnp.concatenate(
        [inds[0, i:i + n] + i * VOCAB for i in range(N_PREDICT)])
    z0, z_rest = _sc_gather(emb_w.reshape(N_PREDICT * VOCAB, INNER_DIM),
                            idx_all, n)

    nr = n // _RB
    m0 = pl.pallas_call(
        _proj0_kernel,
        grid=(nr,),
        in_specs=[
            pl.BlockSpec((1, _RB, EMB_DIM), lambda r: (0, r, 0)),
            pl.BlockSpec((INNER_DIM, EMB_DIM), lambda r: (0, 0)),
        ],
        out_specs=pl.BlockSpec((_RB, INNER_DIM), lambda r: (r, 0)),
        out_shape=jax.ShapeDtypeStruct((n, INNER_DIM), jnp.float32),
        scratch_shapes=[pltpu.VMEM((INNER_DIM, EMB_DIM), jnp.bfloat16)],
        compiler_params=pltpu.CompilerParams(vmem_limit_bytes=100 * 2 ** 20),
    )(state, proj0_w)

    h0 = pl.pallas_call(
        _h0_kernel,
        grid=(nr,),
        in_specs=[
            pl.BlockSpec((_RB, INNER_DIM), lambda r: (r, 0)),
            pl.BlockSpec((_RB, INNER_DIM), lambda r: (r, 0)),
            pl.BlockSpec((N_PREDICT, INNER_DIM), lambda r: (0, 0)),
            pl.BlockSpec((N_PREDICT, INNER_DIM), lambda r: (0, 0)),
        ],
        out_specs=pl.BlockSpec((_RB, INNER_DIM), lambda r: (r, 0)),
        out_shape=jax.ShapeDtypeStruct((n, INNER_DIM), jnp.bfloat16),
        compiler_params=pltpu.CompilerParams(vmem_limit_bytes=100 * 2 ** 20),
    )(m0, z0, ln_w, ln_b)

    nv = VOCAB // _VB
    logits = pl.pallas_call(
        _logits_kernel,
        grid=(N_PREDICT, nv),
        in_specs=[
            pl.BlockSpec(memory_space=pl.ANY),
            pl.BlockSpec(memory_space=pl.ANY),
            pl.BlockSpec(memory_space=pl.ANY),
            pl.BlockSpec((N_PREDICT, INNER_DIM), lambda h, v: (0, 0)),
            pl.BlockSpec((N_PREDICT, INNER_DIM), lambda h, v: (0, 0)),
            pl.BlockSpec((1, _VB, INNER_DIM), lambda h, v: (h, v, 0)),
        ],
        out_specs=pl.BlockSpec((1, 1, n, _VB), lambda h, v: (h, 0, 0, v)),
        out_shape=jax.ShapeDtypeStruct((N_PREDICT, 1, n, VOCAB), jnp.float32),
        scratch_shapes=[
            pltpu.VMEM((2, n, INNER_DIM), jnp.bfloat16),
            pltpu.VMEM((n, INNER_DIM), jnp.float32),
            pltpu.VMEM((INNER_DIM, INNER_DIM), jnp.float32),
            pltpu.SemaphoreType.DMA,
            pltpu.SemaphoreType.DMA,
            pltpu.SemaphoreType.DMA,
        ],
        compiler_params=pltpu.CompilerParams(vmem_limit_bytes=100 * 2 ** 20),
    )(h0, z_rest, proj_rest_w, ln_w, ln_b, head_w)
    return logits


# speedup vs baseline: 1.0272x; 1.0178x over previous
import functools
import math

import jax
import jax.numpy as jnp
from jax import lax
from jax.experimental import pallas as pl
from jax.experimental.pallas import tpu as pltpu
from jax.experimental.pallas import tpu_sc as plsc

N_PREDICT = 3
EMB_DIM = 4096
INNER_DIM = 1024
VOCAB = 32000
SEQ = 2048
STATE_WEIGHT = 0.5 ** (0.5 / N_PREDICT)
EMB_WEIGHT = math.sqrt((1.0 - STATE_WEIGHT ** 2) * (INNER_DIM / 2.0))
ALPHA = EMB_WEIGHT / STATE_WEIGHT
EPS = 1e-06

_RB = 512
_VB = 1280


def _rms(x):
    return x * jax.lax.rsqrt(jnp.mean(x * x, axis=-1, keepdims=True) + EPS)


def _gelu(x):
    return x * 0.5 * (1.0 + jax.lax.erf(x * (0.5 ** 0.5)))


def _proj0_kernel(state_ref, p0_ref, m0_ref, p0_bf):
    @pl.when(pl.program_id(0) == 0)
    def _cast_weights():
        p0_bf[...] = p0_ref[...].astype(jnp.bfloat16)

    x = state_ref[0]
    s = _rms(x) * (0.5 ** 0.5)
    m0_ref[...] = jax.lax.dot_general(s.astype(jnp.bfloat16), p0_bf[...],
                                      (((1,), (1,)), ((), ())),
                                      preferred_element_type=jnp.float32)


def _chain_kernel(m0_ref, z_ref, pr_ref, lnw_ref, lnb_ref, h_ref, pr_bf):
    @pl.when(pl.program_id(0) == 0)
    def _cast_weights():
        pr_bf[...] = pr_ref[...].astype(jnp.bfloat16)

    s = None
    for i in range(N_PREDICT):
        if i == 0:
            m = m0_ref[...]
        else:
            m = jax.lax.dot_general(s.astype(jnp.bfloat16), pr_bf[i - 1],
                                    (((1,), (1,)), ((), ())),
                                    preferred_element_type=jnp.float32)
        t = m + ALPHA * z_ref[i]
        t = _rms(t) * lnw_ref[i][None, :] + lnb_ref[i][None, :]
        s = _gelu(t)
        h_ref[i] = s.astype(jnp.bfloat16)


def _logits_kernel(h_ref, w_ref, out_ref):
    h = h_ref[0]
    w = w_ref[0].astype(jnp.bfloat16)
    out_ref[0, 0] = jax.lax.dot_general(
        h, w, (((1,), (1,)), ((), ())), preferred_element_type=jnp.float32)


def _sc_gather(emb_flat, idx_all, n):
    info = plsc.get_sparse_core_info()
    nc, ns = info.num_cores, info.num_subcores
    nw = nc * ns
    rows_w = n // nw

    mesh = plsc.VectorSubcoreMesh(core_axis_name="c", subcore_axis_name="s")

    @functools.partial(
        pl.kernel,
        out_type=jax.ShapeDtypeStruct((N_PREDICT * n, INNER_DIM), jnp.float32),
        mesh=mesh,
        scratch_types=[
            pltpu.VMEM((rows_w,), jnp.int32),
            pltpu.VMEM((rows_w, INNER_DIM), jnp.float32),
            pltpu.SemaphoreType.DMA,
        ],
    )
    def gather_k(table_hbm, idx_hbm, out_hbm, idx_v, rows_v, sem):
        wid = lax.axis_index("s") * nc + lax.axis_index("c")
        for i in range(N_PREDICT):
            base = i * n + wid * rows_w
            pltpu.sync_copy(idx_hbm.at[pl.ds(base, rows_w)], idx_v)
            pltpu.async_copy(table_hbm.at[idx_v], rows_v, sem).wait()
            pltpu.sync_copy(rows_v, out_hbm.at[pl.ds(base, rows_w)])

    return gather_k(emb_flat, idx_all)


def kernel(state, inds, emb_w, proj0_w, proj_rest_w, head_w, ln_w, ln_b):
    n = state.shape[1]
    idx_all = jnp.concatenate(
        [inds[0, i:i + n] + i * VOCAB for i in range(N_PREDICT)])
    z = _sc_gather(emb_w.reshape(N_PREDICT * VOCAB, INNER_DIM), idx_all, n)
    z = z.reshape(N_PREDICT, n, INNER_DIM)

    nr = n // _RB
    m0 = pl.pallas_call(
        _proj0_kernel,
        grid=(nr,),
        in_specs=[
            pl.BlockSpec((1, _RB, EMB_DIM), lambda r: (0, r, 0)),
            pl.BlockSpec((INNER_DIM, EMB_DIM), lambda r: (0, 0)),
        ],
        out_specs=pl.BlockSpec((_RB, INNER_DIM), lambda r: (r, 0)),
        out_shape=jax.ShapeDtypeStruct((n, INNER_DIM), jnp.float32),
        scratch_shapes=[pltpu.VMEM((INNER_DIM, EMB_DIM), jnp.bfloat16)],
        compiler_params=pltpu.CompilerParams(vmem_limit_bytes=100 * 2 ** 20),
    )(state, proj0_w)

    h_all = pl.pallas_call(
        _chain_kernel,
        grid=(nr,),
        in_specs=[
            pl.BlockSpec((_RB, INNER_DIM), lambda r: (r, 0)),
            pl.BlockSpec((N_PREDICT, _RB, INNER_DIM), lambda r: (0, r, 0)),
            pl.BlockSpec((N_PREDICT - 1, INNER_DIM, INNER_DIM), lambda r: (0, 0, 0)),
            pl.BlockSpec((N_PREDICT, INNER_DIM), lambda r: (0, 0)),
            pl.BlockSpec((N_PREDICT, INNER_DIM), lambda r: (0, 0)),
        ],
        out_specs=pl.BlockSpec((N_PREDICT, _RB, INNER_DIM), lambda r: (0, r, 0)),
        out_shape=jax.ShapeDtypeStruct((N_PREDICT, n, INNER_DIM), jnp.bfloat16),
        scratch_shapes=[
            pltpu.VMEM((N_PREDICT - 1, INNER_DIM, INNER_DIM), jnp.bfloat16),
        ],
        compiler_params=pltpu.CompilerParams(vmem_limit_bytes=100 * 2 ** 20),
    )(m0, z, proj_rest_w, ln_w, ln_b)

    nv = VOCAB // _VB
    logits = pl.pallas_call(
        _logits_kernel,
        grid=(N_PREDICT, nv),
        in_specs=[
            pl.BlockSpec((1, n, INNER_DIM), lambda h, v: (h, 0, 0)),
            pl.BlockSpec((1, _VB, INNER_DIM), lambda h, v: (h, v, 0)),
        ],
        out_specs=pl.BlockSpec((1, 1, n, _VB), lambda h, v: (h, 0, 0, v)),
        out_shape=jax.ShapeDtypeStruct((N_PREDICT, 1, n, VOCAB), jnp.float32),
        compiler_params=pltpu.CompilerParams(vmem_limit_bytes=100 * 2 ** 20),
    )(h_all, head_w)
    return logits
